# baseline (speedup 1.0000x reference)
import functools

import jax
import jax.numpy as jnp
from jax import lax
from jax.experimental import pallas as pl
from jax.experimental.pallas import tpu as pltpu
from jax.experimental.pallas import tpu_sc as plsc

VOCAB = 1000000
EMBED = 64
BATCH = 4096

_QROWS = VOCAB // 4
_QW = 4 * EMBED // 2

_NC = 2
_NS = 16
_NW = _NC * _NS
_BPW = BATCH // _NW

_mesh = plsc.VectorSubcoreMesh(core_axis_name="c", subcore_axis_name="s")


@functools.partial(
    pl.kernel,
    mesh=_mesh,
    out_type=[
        jax.ShapeDtypeStruct((BATCH, _QW), jnp.float32),
        jax.ShapeDtypeStruct((BATCH, _QW), jnp.float32),
    ],
    scratch_types=[
        pltpu.VMEM((_BPW,), jnp.int32),
        pltpu.VMEM((_BPW,), jnp.int32),
        pltpu.VMEM((_BPW, _QW), jnp.float32),
        pltpu.VMEM((_BPW, _QW), jnp.float32),
        pltpu.SemaphoreType.DMA,
        pltpu.SemaphoreType.DMA,
    ],
)
def _sc_gather(tgt_hbm, ctx_hbm, in_tab, out_tab, in_rows_hbm, out_rows_hbm,
               idx_t, idx_c, rows_t, rows_c, sem_t, sem_c):
    wid = lax.axis_index("s") * _NC + lax.axis_index("c")
    base = wid * _BPW
    pltpu.sync_copy(tgt_hbm.at[pl.ds(base, _BPW)], idx_t)
    pltpu.sync_copy(ctx_hbm.at[pl.ds(base, _BPW)], idx_c)

    def fire(g, carry):
        b = g * 16
        vt = idx_t[pl.ds(b, 16)]
        vc = idx_c[pl.ds(b, 16)]
        for j in range(16):
            pltpu.make_async_copy(in_tab.at[vt[j]], rows_t.at[b + j], sem_t).start()
            pltpu.make_async_copy(out_tab.at[vc[j]], rows_c.at[b + j], sem_c).start()
        return carry

    lax.fori_loop(0, _BPW // 16, fire, 0)
    pltpu.make_async_copy(in_tab.at[pl.ds(0, _BPW)], rows_t, sem_t).wait()
    pltpu.make_async_copy(out_tab.at[pl.ds(0, _BPW)], rows_c, sem_c).wait()

    pltpu.sync_copy(rows_t, in_rows_hbm.at[pl.ds(base, _BPW)])
    pltpu.sync_copy(rows_c, out_rows_hbm.at[pl.ds(base, _BPW)])


_BM = 256


def _matmul_body(a_ref, b_ref, o_ref):
    o_ref[...] = lax.dot_general(
        a_ref[...], b_ref[...],
        (((1,), (1,)), ((), ())),
        preferred_element_type=jnp.float32,
    )


_matmul = pl.pallas_call(
    _matmul_body,
    grid=(BATCH // _BM,),
    in_specs=[
        pl.BlockSpec((_BM, EMBED), lambda i: (i, 0)),
        pl.BlockSpec((BATCH, EMBED), lambda i: (0, 0)),
    ],
    out_specs=pl.BlockSpec((_BM, BATCH), lambda i: (i, 0)),
    out_shape=jax.ShapeDtypeStruct((BATCH, BATCH), jnp.float32),
)


def _quad_pack(tab):
    b = tab.astype(jnp.bfloat16).reshape(_QROWS, _QW, 2)
    return jax.lax.bitcast_convert_type(b, jnp.float32)


def _unpack_select(rows, sub):
    b = jax.lax.bitcast_convert_type(rows, jnp.bfloat16)
    b = b.reshape(BATCH, 4, EMBED)
    return jnp.take_along_axis(b, sub[:, None, None], axis=1)[:, 0, :]


def kernel(target, context, in_embed, out_embed):
    target = target.astype(jnp.int32)
    context = context.astype(jnp.int32)
    in_rows, out_rows = _sc_gather(
        jnp.right_shift(target, 2), jnp.right_shift(context, 2),
        _quad_pack(in_embed), _quad_pack(out_embed),
    )
    in_emb = _unpack_select(in_rows, jnp.bitwise_and(target, 3))
    out_emb = _unpack_select(out_rows, jnp.bitwise_and(context, 3))
    return _matmul(in_emb, out_emb)


# speedup vs baseline: 38.4748x; 38.4748x over previous
import functools

import jax
import jax.numpy as jnp
from jax import lax
from jax.experimental import pallas as pl
from jax.experimental.pallas import tpu as pltpu
from jax.experimental.pallas import tpu_sc as plsc

VOCAB = 1000000
EMBED = 64
BATCH = 4096

_QROWS = VOCAB // 2
_QW = 2 * EMBED

_NC = 2
_NS = 16
_NW = _NC * _NS
_BPW = BATCH // _NW

_mesh = plsc.VectorSubcoreMesh(core_axis_name="c", subcore_axis_name="s")


@functools.partial(
    pl.kernel,
    mesh=_mesh,
    out_type=[
        jax.ShapeDtypeStruct((BATCH, _QW), jnp.float32),
        jax.ShapeDtypeStruct((BATCH, _QW), jnp.float32),
    ],
    scratch_types=[
        pltpu.VMEM((_BPW,), jnp.int32),
        pltpu.VMEM((_BPW,), jnp.int32),
        pltpu.VMEM((_BPW, _QW), jnp.float32),
        pltpu.VMEM((_BPW, _QW), jnp.float32),
        pltpu.SemaphoreType.DMA,
        pltpu.SemaphoreType.DMA,
    ],
)
def _sc_gather(tgt_hbm, ctx_hbm, in_tab, out_tab, in_rows_hbm, out_rows_hbm,
               idx_t, idx_c, rows_t, rows_c, sem_t, sem_c):
    wid = lax.axis_index("s") * _NC + lax.axis_index("c")
    base = wid * _BPW
    pltpu.sync_copy(tgt_hbm.at[pl.ds(base, _BPW)], idx_t)
    pltpu.sync_copy(ctx_hbm.at[pl.ds(base, _BPW)], idx_c)

    def fire(g, carry):
        b = g * 16
        vt = idx_t[pl.ds(b, 16)]
        vc = idx_c[pl.ds(b, 16)]
        for j in range(16):
            pltpu.make_async_copy(in_tab.at[vt[j]], rows_t.at[b + j], sem_t).start()
            pltpu.make_async_copy(out_tab.at[vc[j]], rows_c.at[b + j], sem_c).start()
        return carry

    lax.fori_loop(0, _BPW // 16, fire, 0)
    pltpu.make_async_copy(in_tab.at[pl.ds(0, _BPW)], rows_t, sem_t).wait()
    pltpu.make_async_copy(out_tab.at[pl.ds(0, _BPW)], rows_c, sem_c).wait()

    pltpu.sync_copy(rows_t, in_rows_hbm.at[pl.ds(base, _BPW)])
    pltpu.sync_copy(rows_c, out_rows_hbm.at[pl.ds(base, _BPW)])


_BM = 256


def _matmul_body(a_ref, b_ref, o_ref):
    o_ref[...] = lax.dot_general(
        a_ref[...], b_ref[...],
        (((1,), (1,)), ((), ())),
        preferred_element_type=jnp.float32,
    )


_matmul = pl.pallas_call(
    _matmul_body,
    grid=(BATCH // _BM,),
    in_specs=[
        pl.BlockSpec((_BM, EMBED), lambda i: (i, 0)),
        pl.BlockSpec((BATCH, EMBED), lambda i: (0, 0)),
    ],
    out_specs=pl.BlockSpec((_BM, BATCH), lambda i: (i, 0)),
    out_shape=jax.ShapeDtypeStruct((BATCH, BATCH), jnp.float32),
)


def _pair_pack(tab):
    return tab.reshape(_QROWS, _QW)


def _unpack_select(rows, sub):
    b = rows.reshape(BATCH, 2, EMBED)
    return jnp.take_along_axis(b, sub[:, None, None], axis=1)[:, 0, :]


def kernel(target, context, in_embed, out_embed):
    target = target.astype(jnp.int32)
    context = context.astype(jnp.int32)
    in_rows, out_rows = _sc_gather(
        jnp.right_shift(target, 1), jnp.right_shift(context, 1),
        _pair_pack(in_embed), _pair_pack(out_embed),
    )
    in_emb = _unpack_select(in_rows, jnp.bitwise_and(target, 1))
    out_emb = _unpack_select(out_rows, jnp.bitwise_and(context, 1))
    return _matmul(in_emb, out_emb)


# speedup vs baseline: 370.3950x; 9.6269x over previous
import functools

import jax
import jax.numpy as jnp
from jax import lax
from jax.experimental import pallas as pl
from jax.experimental.pallas import tpu as pltpu

VOCAB = 1000000
EMBED = 64
BATCH = 4096

_IBLK = 128
_NSTEP = BATCH // _IBLK
_W = 128
_STACK = _IBLK * _W


def _gather_body(tgt_smem, ctx_smem, tgt_ref, ctx_ref, in_tab, out_tab,
                 oa_ref, ob_ref, stk_a, stk_b, sem_a, sem_b):
    s = pl.program_id(0)

    def fire(step, buf):
        for j in range(_IBLK):
            wt = pl.multiple_of((tgt_smem[step, j] >> 7) * _W, _W)
            wc = pl.multiple_of((ctx_smem[step, j] >> 7) * _W, _W)
            pltpu.make_async_copy(
                in_tab.at[:, pl.ds(wt, _W)],
                stk_a.at[buf, :, pl.ds(j * _W, _W)], sem_a.at[buf]).start()
            pltpu.make_async_copy(
                out_tab.at[:, pl.ds(wc, _W)],
                stk_b.at[buf, :, pl.ds(j * _W, _W)], sem_b.at[buf]).start()

    @pl.when(s == 0)
    def _():
        fire(0, 0)

    @pl.when(s + 1 < _NSTEP)
    def _():
        fire(s + 1, (s + 1) % 2)

    buf = s % 2
    pltpu.make_async_copy(in_tab.at[:, pl.ds(0, _STACK)], stk_a.at[buf],
                          sem_a.at[buf]).wait()
    pltpu.make_async_copy(out_tab.at[:, pl.ds(0, _STACK)], stk_b.at[buf],
                          sem_b.at[buf]).wait()

    li = lax.broadcasted_iota(jnp.int32, (_STACK, _IBLK), 0)
    ji = lax.broadcasted_iota(jnp.int32, (_STACK, _IBLK), 1)
    rt = (tgt_ref[0, 0, :] & 127)[None, :]
    rc = (ctx_ref[0, 0, :] & 127)[None, :]
    sel_t = (li == ji * _W + rt).astype(jnp.float32)
    sel_c = (li == ji * _W + rc).astype(jnp.float32)
    oa_ref[...] = lax.dot_general(
        stk_a[buf], sel_t, (((1,), (0,)), ((), ())),
        preferred_element_type=jnp.float32)
    ob_ref[...] = lax.dot_general(
        stk_b[buf], sel_c, (((1,), (0,)), ((), ())),
        preferred_element_type=jnp.float32)


_gather = pl.pallas_call(
    _gather_body,
    grid_spec=pltpu.PrefetchScalarGridSpec(
        num_scalar_prefetch=2,
        grid=(_NSTEP,),
        in_specs=[
            pl.BlockSpec((1, 1, _IBLK), lambda s, tgt, ctx: (s, 0, 0)),
            pl.BlockSpec((1, 1, _IBLK), lambda s, tgt, ctx: (s, 0, 0)),
            pl.BlockSpec(memory_space=pl.ANY),
            pl.BlockSpec(memory_space=pl.ANY),
        ],
        out_specs=[
            pl.BlockSpec((EMBED, _IBLK), lambda s, tgt, ctx: (0, s)),
            pl.BlockSpec((EMBED, _IBLK), lambda s, tgt, ctx: (0, s)),
        ],
        scratch_shapes=[
            pltpu.VMEM((2, EMBED, _STACK), jnp.float32),
            pltpu.VMEM((2, EMBED, _STACK), jnp.float32),
            pltpu.SemaphoreType.DMA((2,)),
            pltpu.SemaphoreType.DMA((2,)),
        ],
    ),
    out_shape=[
        jax.ShapeDtypeStruct((EMBED, BATCH), jnp.float32),
        jax.ShapeDtypeStruct((EMBED, BATCH), jnp.float32),
    ],
)


_BM = 256


def _matmul_body(a_ref, b_ref, o_ref):
    o_ref[...] = lax.dot_general(
        a_ref[...], b_ref[...],
        (((0,), (0,)), ((), ())),
        preferred_element_type=jnp.float32,
    )


_matmul = pl.pallas_call(
    _matmul_body,
    grid=(BATCH // _BM,),
    in_specs=[
        pl.BlockSpec((EMBED, _BM), lambda i: (0, i)),
        pl.BlockSpec((EMBED, BATCH), lambda i: (0, 0)),
    ],
    out_specs=pl.BlockSpec((_BM, BATCH), lambda i: (i, 0)),
    out_shape=jax.ShapeDtypeStruct((BATCH, BATCH), jnp.float32),
)


def kernel(target, context, in_embed, out_embed):
    target = target.astype(jnp.int32).reshape(_NSTEP, _IBLK)
    context = context.astype(jnp.int32).reshape(_NSTEP, _IBLK)
    a_t, b_t = _gather(target, context,
                       target[:, None, :], context[:, None, :],
                       in_embed.T, out_embed.T)
    return _matmul(a_t, b_t)
